# baseline (speedup 1.0000x reference)
import jax
import jax.numpy as jnp
from jax.experimental import pallas as pl
from jax.experimental.pallas import tpu as pltpu

_K = 8192
_B = 8
_P = 96
_W = 224
_PCHUNK = 12
_NCHUNK = _P // _PCHUNK
_SLABS = 8

_TOPBIT = 0x7FFFFFFF


def _abs_bits(x):
    return jax.lax.bitcast_convert_type(x, jnp.int32) & jnp.int32(_TOPBIT)


def _count_gt(x_ref, mid):
    s = _P // _SLABS

    def slab(i, acc):
        bits = _abs_bits(x_ref[0, pl.ds(i * s, s)])
        return acc + jnp.sum((bits > mid).astype(jnp.int32))

    return jax.lax.fori_loop(0, _SLABS, slab, jnp.int32(0))


def _count3_gt(x_ref, m1, m2, m3):
    s = _P // _SLABS

    def slab(i, accs):
        a1, a2, a3 = accs
        bits = _abs_bits(x_ref[0, pl.ds(i * s, s)])
        return (a1 + jnp.sum((bits > m1).astype(jnp.int32)),
                a2 + jnp.sum((bits > m2).astype(jnp.int32)),
                a3 + jnp.sum((bits > m3).astype(jnp.int32)))

    z = jnp.int32(0)
    return jax.lax.fori_loop(0, _SLABS, slab, (z, z, z))


def _kth_largest(x_ref, k, lo0, hi0, c_hi0):
    def cond(carry):
        lo, hi, _ = carry
        return lo < hi

    def body(carry):
        lo, hi, c_hi = carry
        span = hi - lo
        m1 = lo + span // 4
        m2 = lo + span // 2
        m3 = lo + (span // 4) * 3
        c1, c2, c3 = _count3_gt(x_ref, m1, m2, m3)
        new_hi = jnp.where(c1 < k, m1,
                           jnp.where(c2 < k, m2,
                                     jnp.where(c3 < k, m3, hi)))
        new_chi = jnp.where(c1 < k, c1,
                            jnp.where(c2 < k, c2,
                                      jnp.where(c3 < k, c3, c_hi)))
        new_lo = jnp.where(c1 < k, lo,
                           jnp.where(c2 < k, m1 + 1,
                                     jnp.where(c3 < k, m2 + 1, m3 + 1)))
        return new_lo, new_hi, new_chi

    _, hi, c_hi = jax.lax.while_loop(cond, body, (lo0, hi0, c_hi0))
    return hi, c_hi


def _thresh_kernel(x_ref, v_ref, need_ref):
    def sbody(_, carry):
        lo1, hi1, lo2, hi2 = carry
        mid1 = lo1 + (hi1 - lo1) // 2
        mid2 = lo2 + (hi2 - lo2) // 2
        sbits = _abs_bits(x_ref[0, 0:1])
        c1 = jnp.sum((sbits > mid1).astype(jnp.int32))
        c2 = jnp.sum((sbits > mid2).astype(jnp.int32))
        t1 = c1 < 34
        t2 = c2 < 137
        return (jnp.where(t1, lo1, mid1 + 1), jnp.where(t1, mid1, hi1),
                jnp.where(t2, lo2, mid2 + 1), jnp.where(t2, mid2, hi2))

    z = jnp.int32(0)
    top = jnp.int32(_TOPBIT)
    _, t_hi, _, t_lo = jax.lax.fori_loop(0, 31, sbody, (z, top, z, top))

    c_hi = _count_gt(x_ref, t_hi)
    c_lo = _count_gt(x_ref, t_lo)
    lo0 = jnp.where(c_lo >= _K, t_lo + 1, z)
    hi0 = jnp.where(c_hi < _K, t_hi, top)
    c_hi0 = jnp.where(c_hi < _K, c_hi, z)

    v, c_above = _kth_largest(x_ref, jnp.int32(_K), lo0, hi0, c_hi0)
    v_ref[...] = jnp.full((1, 1, 1), v, jnp.int32)
    need_ref[...] = jnp.full((1, 1, 1), _K - c_above, jnp.int32)


def _mask_kernel(v_ref, need_ref, x_ref, o_ref, carry_ref):
    c = pl.program_id(1)

    @pl.when(c == 0)
    def _():
        carry_ref[0] = jnp.int32(0)

    v = v_ref[0, 0, 0]
    need = need_ref[0, 0, 0]
    x = x_ref[0]
    bits = _abs_bits(x)
    eq = bits == v
    gt = bits > v
    eq_cnt = jnp.sum(eq.astype(jnp.int32))

    @pl.when(eq_cnt == 0)
    def _():
        o_ref[...] = jnp.where(gt, x, jnp.float32(0.0))[None]

    @pl.when(eq_cnt > 0)
    def _():
        eqb = eq.astype(jnp.bfloat16)

        row_i = jax.lax.broadcasted_iota(jnp.int32, (_W, _W), 0)
        col_i = jax.lax.broadcasted_iota(jnp.int32, (_W, _W), 1)
        u224 = (row_i < col_i).astype(jnp.bfloat16)

        p_lane = jax.lax.dot_general(
            eqb, u224, (((2,), (0,)), ((), ())),
            preferred_element_type=jnp.float32)

        row_sums = jnp.sum(eqb.astype(jnp.float32), axis=2)
        p_row = jax.lax.dot_general(
            row_sums.astype(jnp.bfloat16), u224, (((1,), (0,)), ((), ())),
            preferred_element_type=jnp.float32)
        plane_tot = jnp.sum(row_sums, axis=1)
        gi = jax.lax.broadcasted_iota(jnp.int32, (_PCHUNK, _PCHUNK), 0)
        gj = jax.lax.broadcasted_iota(jnp.int32, (_PCHUNK, _PCHUNK), 1)
        plane_pref = jnp.sum(
            jnp.where(gj < gi, plane_tot[None, :], 0.0), axis=1)

        rank = (p_lane
                + (p_row + plane_pref[:, None])[:, :, None]
                + carry_ref[0].astype(jnp.float32))

        keep = gt | (eq & (rank < need.astype(jnp.float32)))
        o_ref[...] = jnp.where(keep, x, jnp.float32(0.0))[None]

    carry_ref[0] += eq_cnt


@jax.jit
def kernel(input):
    v, need = pl.pallas_call(
        _thresh_kernel,
        grid=(_B,),
        in_specs=[pl.BlockSpec((1, _P, _W, _W), lambda b: (b, 0, 0, 0))],
        out_specs=[pl.BlockSpec((1, 1, 1), lambda b: (b, 0, 0)),
                   pl.BlockSpec((1, 1, 1), lambda b: (b, 0, 0))],
        out_shape=[jax.ShapeDtypeStruct((_B, 1, 1), jnp.int32),
                   jax.ShapeDtypeStruct((_B, 1, 1), jnp.int32)],
    )(input)

    out = pl.pallas_call(
        _mask_kernel,
        grid=(_B, _NCHUNK),
        in_specs=[
            pl.BlockSpec((1, 1, 1), lambda b, c: (b, 0, 0)),
            pl.BlockSpec((1, 1, 1), lambda b, c: (b, 0, 0)),
            pl.BlockSpec((1, _PCHUNK, _W, _W), lambda b, c: (b, c, 0, 0)),
        ],
        out_specs=pl.BlockSpec((1, _PCHUNK, _W, _W), lambda b, c: (b, c, 0, 0)),
        out_shape=jax.ShapeDtypeStruct((_B, _P, _W, _W), jnp.float32),
        scratch_shapes=[pltpu.SMEM((1,), jnp.int32)],
    )(v, need, input)

    return out


# speedup vs baseline: 1.0814x; 1.0814x over previous
import jax
import jax.numpy as jnp
from jax.experimental import pallas as pl
from jax.experimental.pallas import tpu as pltpu

_K = 8192
_B = 8
_P = 96
_W = 224
_PCHUNK = 24
_NCHUNK = _P // _PCHUNK
_SLABS = 8

_TOPBIT = 0x7FFFFFFF


def _abs_bits(x):
    return jax.lax.bitcast_convert_type(x, jnp.int32) & jnp.int32(_TOPBIT)


def _count_gt(x_ref, mid):
    s = _P // _SLABS

    def slab(i, acc):
        bits = _abs_bits(x_ref[0, pl.ds(i * s, s)])
        return acc + jnp.sum((bits > mid).astype(jnp.int32))

    return jax.lax.fori_loop(0, _SLABS, slab, jnp.int32(0))


def _count2_gt(x_ref, m1, m2):
    s = _P // _SLABS

    def slab(i, accs):
        a1, a2 = accs
        bits = _abs_bits(x_ref[0, pl.ds(i * s, s)])
        return (a1 + jnp.sum((bits > m1).astype(jnp.int32)),
                a2 + jnp.sum((bits > m2).astype(jnp.int32)))

    z = jnp.int32(0)
    return jax.lax.fori_loop(0, _SLABS, slab, (z, z))


def _kth_largest(x_ref, k, lo0, hi0, c_hi0):
    def cond(carry):
        lo, hi, _ = carry
        return lo < hi

    def body(carry):
        lo, hi, c_hi = carry
        mid = lo + (hi - lo) // 2
        c = _count_gt(x_ref, mid)
        take_low = c < k
        return (jnp.where(take_low, lo, mid + 1),
                jnp.where(take_low, mid, hi),
                jnp.where(take_low, c, c_hi))

    _, hi, c_hi = jax.lax.while_loop(cond, body, (lo0, hi0, c_hi0))
    return hi, c_hi


def _thresh_kernel(x_ref, v_ref, need_ref):
    def sbody(_, carry):
        lo1, hi1, lo2, hi2 = carry
        mid1 = lo1 + (hi1 - lo1) // 2
        mid2 = lo2 + (hi2 - lo2) // 2
        sbits = _abs_bits(x_ref[0, 0:1])
        c1 = jnp.sum((sbits > mid1).astype(jnp.int32))
        c2 = jnp.sum((sbits > mid2).astype(jnp.int32))
        t1 = c1 < 34
        t2 = c2 < 137
        return (jnp.where(t1, lo1, mid1 + 1), jnp.where(t1, mid1, hi1),
                jnp.where(t2, lo2, mid2 + 1), jnp.where(t2, mid2, hi2))

    z = jnp.int32(0)
    top = jnp.int32(_TOPBIT)
    _, t_hi, _, t_lo = jax.lax.fori_loop(0, 31, sbody, (z, top, z, top))

    c_hi, c_lo = _count2_gt(x_ref, t_hi, t_lo)
    lo0 = jnp.where(c_lo >= _K, t_lo + 1, z)
    hi0 = jnp.where(c_hi < _K, t_hi, top)
    c_hi0 = jnp.where(c_hi < _K, c_hi, z)

    v, c_above = _kth_largest(x_ref, jnp.int32(_K), lo0, hi0, c_hi0)
    v_ref[...] = jnp.full((1, 1, 1), v, jnp.int32)
    need_ref[...] = jnp.full((1, 1, 1), _K - c_above, jnp.int32)


def _mask_kernel(v_ref, need_ref, x_ref, o_ref, carry_ref):
    c = pl.program_id(1)

    @pl.when(c == 0)
    def _():
        carry_ref[0] = jnp.int32(0)

    v = v_ref[0, 0, 0]
    need = need_ref[0, 0, 0]
    x = x_ref[0]
    bits = _abs_bits(x)
    eq = bits == v
    gt = bits > v
    eq_cnt = jnp.sum(eq.astype(jnp.int32))

    @pl.when(eq_cnt == 0)
    def _():
        o_ref[...] = jnp.where(gt, x, jnp.float32(0.0))[None]

    @pl.when(eq_cnt > 0)
    def _():
        eqb = eq.astype(jnp.bfloat16)

        row_i = jax.lax.broadcasted_iota(jnp.int32, (_W, _W), 0)
        col_i = jax.lax.broadcasted_iota(jnp.int32, (_W, _W), 1)
        u224 = (row_i < col_i).astype(jnp.bfloat16)

        p_lane = jax.lax.dot_general(
            eqb, u224, (((2,), (0,)), ((), ())),
            preferred_element_type=jnp.float32)

        row_sums = jnp.sum(eqb.astype(jnp.float32), axis=2)
        p_row = jax.lax.dot_general(
            row_sums.astype(jnp.bfloat16), u224, (((1,), (0,)), ((), ())),
            preferred_element_type=jnp.float32)
        plane_tot = jnp.sum(row_sums, axis=1)
        gi = jax.lax.broadcasted_iota(jnp.int32, (_PCHUNK, _PCHUNK), 0)
        gj = jax.lax.broadcasted_iota(jnp.int32, (_PCHUNK, _PCHUNK), 1)
        plane_pref = jnp.sum(
            jnp.where(gj < gi, plane_tot[None, :], 0.0), axis=1)

        rank = (p_lane
                + (p_row + plane_pref[:, None])[:, :, None]
                + carry_ref[0].astype(jnp.float32))

        keep = gt | (eq & (rank < need.astype(jnp.float32)))
        o_ref[...] = jnp.where(keep, x, jnp.float32(0.0))[None]

    carry_ref[0] += eq_cnt


@jax.jit
def kernel(input):
    v, need = pl.pallas_call(
        _thresh_kernel,
        grid=(_B,),
        in_specs=[pl.BlockSpec((1, _P, _W, _W), lambda b: (b, 0, 0, 0))],
        out_specs=[pl.BlockSpec((1, 1, 1), lambda b: (b, 0, 0)),
                   pl.BlockSpec((1, 1, 1), lambda b: (b, 0, 0))],
        out_shape=[jax.ShapeDtypeStruct((_B, 1, 1), jnp.int32),
                   jax.ShapeDtypeStruct((_B, 1, 1), jnp.int32)],
    )(input)

    out = pl.pallas_call(
        _mask_kernel,
        grid=(_B, _NCHUNK),
        in_specs=[
            pl.BlockSpec((1, 1, 1), lambda b, c: (b, 0, 0)),
            pl.BlockSpec((1, 1, 1), lambda b, c: (b, 0, 0)),
            pl.BlockSpec((1, _PCHUNK, _W, _W), lambda b, c: (b, c, 0, 0)),
        ],
        out_specs=pl.BlockSpec((1, _PCHUNK, _W, _W), lambda b, c: (b, c, 0, 0)),
        out_shape=jax.ShapeDtypeStruct((_B, _P, _W, _W), jnp.float32),
        scratch_shapes=[pltpu.SMEM((1,), jnp.int32)],
    )(v, need, input)

    return out
